# baseline (speedup 1.0000x reference)
import jax
import jax.numpy as jnp
from jax.experimental import pallas as pl
from jax.experimental.pallas import tpu as pltpu

FEAT = 115
EMB = 13
T = 13
OUT = FEAT + EMB

BLOCK_N = 1250


def _concat_kernel(data_ref, embp_ref, out_ref):
    d = data_ref[...]
    dpad = jnp.pad(d, ((0, 0), (0, 0), (0, 0), (0, EMB)))
    out_ref[...] = dpad + embp_ref[...]


def kernel(data, emb_table):
    g, n, t, f = data.shape
    embp = jnp.pad(emb_table, ((0, 0), (FEAT, 0)))[None, None]

    return pl.pallas_call(
        _concat_kernel,
        grid=(g, n // BLOCK_N),
        in_specs=[
            pl.BlockSpec((1, BLOCK_N, t, f), lambda gi, i: (gi, i, 0, 0)),
            pl.BlockSpec((1, 1, t, OUT), lambda gi, i: (0, 0, 0, 0)),
        ],
        out_specs=pl.BlockSpec((1, BLOCK_N, t, OUT), lambda gi, i: (gi, i, 0, 0)),
        out_shape=jax.ShapeDtypeStruct((g, n, t, OUT), data.dtype),
        compiler_params=pltpu.CompilerParams(
            dimension_semantics=("parallel", "parallel")),
    )(data, embp)


# speedup vs baseline: 1.0021x; 1.0021x over previous
import jax
import jax.numpy as jnp
from jax.experimental import pallas as pl
from jax.experimental.pallas import tpu as pltpu

FEAT = 115
EMB = 13
T = 13
OUT = FEAT + EMB

BLOCK_N = 1000


def _concat_kernel(data_ref, embp_ref, out_ref):
    d = data_ref[...]
    dpad = jnp.pad(d, ((0, 0), (0, 0), (0, 0), (0, EMB)))
    out_ref[...] = dpad + embp_ref[...]


def kernel(data, emb_table):
    g, n, t, f = data.shape
    embp = jnp.pad(emb_table, ((0, 0), (FEAT, 0)))[None, None]

    return pl.pallas_call(
        _concat_kernel,
        grid=(g, n // BLOCK_N),
        in_specs=[
            pl.BlockSpec((1, BLOCK_N, t, f), lambda gi, i: (gi, i, 0, 0)),
            pl.BlockSpec((1, 1, t, OUT), lambda gi, i: (0, 0, 0, 0)),
        ],
        out_specs=pl.BlockSpec((1, BLOCK_N, t, OUT), lambda gi, i: (gi, i, 0, 0)),
        out_shape=jax.ShapeDtypeStruct((g, n, t, OUT), data.dtype),
        compiler_params=pltpu.CompilerParams(
            dimension_semantics=("parallel", "parallel")),
    )(data, embp)
